# baseline (speedup 1.0000x reference)
import jax
import jax.numpy as jnp
from jax import lax
from jax.experimental import pallas as pl
from jax.experimental.pallas import tpu as pltpu
from jax.experimental.pallas import tpu_sc as plsc

_B, _L, _H = 1024, 200, 64
_NC, _NS = 2, 16
_NW = _NC * _NS
_CH = 64
_TOK = _B * _L
_GCH = _TOK // _CH
_NCH = _GCH // _NW


_VB = 7812
_NKF = 244


def _fmt_body(tt_hbm, tail_hbm, out_hbm, vbuf, obuf, osem2):
    wid = lax.axis_index("s") * _NC + lax.axis_index("c")

    def do_block(k, carry):
        tc = wid + k * _NW
        s = k % 2
        c0 = pl.multiple_of(tc * 128, 128)

        @pl.when(k >= 2)
        def _():
            tp = wid + (k - 2) * _NW
            pltpu.make_async_copy(obuf.at[s], out_hbm.at[pl.ds(tp * 64, 64)],
                                  osem2).wait()

        pltpu.sync_copy(tt_hbm.at[:, pl.ds(c0, 128)], vbuf.at[s])

        def col_body(vc, carry2):
            ivc = jnp.full((16,), vc, jnp.int32)
            r = vc // 2
            cb = (vc % 2) * 64
            for d in range(4):
                ih = lax.iota(jnp.int32, 16) + (16 * d)
                vals = plsc.load_gather(vbuf.at[s], [ih, ivc])
                obuf[s, r, pl.ds(cb + 16 * d, 16)] = vals
            return carry2

        lax.fori_loop(0, 128, col_body, 0)
        pltpu.async_copy(obuf.at[s], out_hbm.at[pl.ds(tc * 64, 64)], osem2)
        return carry

    lax.fori_loop(0, _NKF, do_block, 0)
    @pl.when(wid < _VB - _NKF * _NW)
    def _():
        do_block(_NKF, 0)

    @pl.when(wid == 4)
    def _():
        pltpu.sync_copy(tail_hbm, vbuf.at[0, pl.ds(0, 32)])
        pltpu.sync_copy(vbuf.at[0, pl.ds(0, 32)], out_hbm.at[pl.ds(_VB * 64, 32)])

    @pl.when(wid < 4)
    def _():
        pltpu.make_async_copy(obuf.at[1], out_hbm.at[pl.ds((wid + 243 * _NW) * 64, 64)], osem2).wait()
        pltpu.make_async_copy(obuf.at[0], out_hbm.at[pl.ds((wid + 244 * _NW) * 64, 64)], osem2).wait()

    @pl.when(wid >= 4)
    def _():
        pltpu.make_async_copy(obuf.at[0], out_hbm.at[pl.ds((wid + 242 * _NW) * 64, 64)], osem2).wait()
        pltpu.make_async_copy(obuf.at[1], out_hbm.at[pl.ds((wid + 243 * _NW) * 64, 64)], osem2).wait()


def _format_table(token_table):
    tail = token_table[_VB * 128:].reshape(32, 128)
    return pl.kernel(
        _fmt_body,
        out_type=jax.ShapeDtypeStruct((500000, 128), jnp.float32),
        mesh=plsc.VectorSubcoreMesh(core_axis_name="c", subcore_axis_name="s"),
        compiler_params=pltpu.CompilerParams(use_tc_tiling_on_sc=True, needs_layout_passes=False),
        scratch_types=[
            pltpu.VMEM((2, 64, 128), jnp.float32),
            pltpu.VMEM((2, 64, 128), jnp.float32),
            pltpu.SemaphoreType.DMA,
        ],
    )(token_table.T, tail)


def _emb_body(ids_hbm, tok_hbm, pos_hbm, out_hbm,
              idx_v, pos_sh, buf_v, obuf_v, gsem, osem):
    sid = lax.axis_index("s")
    wid = sid * _NC + lax.axis_index("c")
    base = wid * _NCH
    pltpu.sync_copy(ids_hbm.at[pl.ds(base, _NCH)], idx_v)

    @pl.when(sid == 0)
    def _():
        pltpu.sync_copy(pos_hbm.at[pl.ds(0, _L)], pos_sh.at[pl.ds(0, _L)])
        pltpu.sync_copy(pos_hbm.at[pl.ds(0, _L)], pos_sh.at[pl.ds(_L, _L)])

    plsc.subcore_barrier()

    def prefill_and_gather(c, slot):
        base_p = (c * _CH) % _L
        pltpu.sync_copy(pos_sh.at[pl.ds(base_p, _CH)], buf_v.at[slot])
        pltpu.async_copy(tok_hbm.at[idx_v.at[c]], buf_v.at[slot], gsem, add=True)

    prefill_and_gather(0, 0)

    def chunk_body(c, carry):
        s = c % 2

        @pl.when(c >= 2)
        def _():
            pltpu.make_async_copy(obuf_v.at[s], out_hbm.at[base + c - 2], osem).wait()

        @pl.when(c + 1 < _NCH)
        def _():
            prefill_and_gather(c + 1, 1 - s)

        pltpu.make_async_copy(tok_hbm.at[idx_v.at[c]], buf_v.at[s], gsem).wait()

        def row_body(t, carry2):
            for d in range(_H // 16):
                sl = pl.ds(d * 16, 16)
                obuf_v[s, t, sl] = buf_v[s, t, sl]
            return carry2

        lax.fori_loop(0, _CH, row_body, 0)

        pltpu.async_copy(obuf_v.at[s], out_hbm.at[base + c], osem)
        return carry

    lax.fori_loop(0, _NCH, chunk_body, 0)
    pltpu.make_async_copy(obuf_v.at[_NCH % 2], out_hbm.at[base + _NCH - 2], osem).wait()
    pltpu.make_async_copy(obuf_v.at[(_NCH - 1) % 2], out_hbm.at[base + _NCH - 1], osem).wait()


@jax.jit
def kernel(input_ids, token_table, pos_table):
    ids = input_ids.reshape(_GCH, _CH).astype(jnp.int32)
    out = pl.kernel(
        _emb_body,
        out_type=jax.ShapeDtypeStruct((_GCH, _CH, 2 * _H), jnp.float32),
        mesh=plsc.VectorSubcoreMesh(core_axis_name="c", subcore_axis_name="s"),
        compiler_params=pltpu.CompilerParams(use_tc_tiling_on_sc=False),
        scratch_types=[
            pltpu.VMEM((_NCH, _CH), jnp.int32),
            pltpu.VMEM_SHARED((2 * _L, _H), jnp.float32),
            pltpu.VMEM((2, _CH, _H), jnp.float32),
            pltpu.VMEM((2, _CH, 2 * _H), jnp.float32),
            pltpu.SemaphoreType.DMA,
            pltpu.SemaphoreType.DMA,
        ],
    )(ids, _format_table(token_table).reshape(1000000, _H), pos_table)
    return out[:, :, :_H].reshape(_B, _L, _H)


# speedup vs baseline: 2.4645x; 2.4645x over previous
import jax
import jax.numpy as jnp
from jax import lax
from jax.experimental import pallas as pl
from jax.experimental.pallas import tpu as pltpu
from jax.experimental.pallas import tpu_sc as plsc

_B, _L, _H = 1024, 200, 64
_NC, _NS = 2, 16
_NW = _NC * _NS
_CH = 64
_TOK = _B * _L
_GCH = _TOK // _CH
_NCH = _GCH // _NW


def _emb_body(ids_hbm, tok_hbm, pos_hbm, out_hbm,
              idx_v, pos_sh, buf_v, obuf_v, gsem, osem):
    sid = lax.axis_index("s")
    wid = sid * _NC + lax.axis_index("c")
    base = wid * _NCH
    pltpu.sync_copy(ids_hbm.at[pl.ds(base, _NCH)], idx_v)

    @pl.when(sid == 0)
    def _():
        pltpu.sync_copy(pos_hbm.at[pl.ds(0, _L)], pos_sh.at[pl.ds(0, _L)])
        pltpu.sync_copy(pos_hbm.at[pl.ds(0, _L)], pos_sh.at[pl.ds(_L, _L)])

    plsc.subcore_barrier()

    def prefill_and_gather(c, slot):
        base_p = (c * _CH) % _L
        pltpu.sync_copy(pos_sh.at[pl.ds(base_p, _CH)], buf_v.at[slot])
        pltpu.async_copy(tok_hbm.at[idx_v.at[c]], buf_v.at[slot], gsem, add=True)

    prefill_and_gather(0, 0)

    def chunk_body(c, carry):
        s = c % 2

        @pl.when(c >= 2)
        def _():
            pltpu.make_async_copy(obuf_v.at[s], out_hbm.at[base + c - 2], osem).wait()

        @pl.when(c + 1 < _NCH)
        def _():
            prefill_and_gather(c + 1, 1 - s)

        pltpu.make_async_copy(tok_hbm.at[idx_v.at[c]], buf_v.at[s], gsem).wait()

        def row_body(t, carry2):
            for d in range(_H // 16):
                sl = pl.ds(d * 16, 16)
                obuf_v[s, t, sl] = buf_v[s, t, sl]
            return carry2

        lax.fori_loop(0, _CH, row_body, 0)

        pltpu.async_copy(obuf_v.at[s], out_hbm.at[base + c], osem)
        return carry

    lax.fori_loop(0, _NCH, chunk_body, 0)
    pltpu.make_async_copy(obuf_v.at[_NCH % 2], out_hbm.at[base + _NCH - 2], osem).wait()
    pltpu.make_async_copy(obuf_v.at[(_NCH - 1) % 2], out_hbm.at[base + _NCH - 1], osem).wait()


@jax.jit
def kernel(input_ids, token_table, pos_table):
    ids = input_ids.reshape(_GCH, _CH).astype(jnp.int32)
    out = pl.kernel(
        _emb_body,
        out_type=jax.ShapeDtypeStruct((_GCH, _CH, 2 * _H), jnp.float32),
        mesh=plsc.VectorSubcoreMesh(core_axis_name="c", subcore_axis_name="s"),
        compiler_params=pltpu.CompilerParams(use_tc_tiling_on_sc=False),
        scratch_types=[
            pltpu.VMEM((_NCH, _CH), jnp.int32),
            pltpu.VMEM_SHARED((2 * _L, _H), jnp.float32),
            pltpu.VMEM((2, _CH, _H), jnp.float32),
            pltpu.VMEM((2, _CH, 2 * _H), jnp.float32),
            pltpu.SemaphoreType.DMA,
            pltpu.SemaphoreType.DMA,
        ],
    )(ids, token_table, pos_table)
    return out[:, :, :_H].reshape(_B, _L, _H)
